# baseline (speedup 1.0000x reference)
import functools

import jax
import jax.numpy as jnp
from jax import lax
from jax.experimental import pallas as pl
from jax.experimental.pallas import tpu as pltpu
from jax.experimental.pallas import tpu_sc as plsc

H = 8
NC = 2
NS = 16
NW = NC * NS
CH_T = 50
CHUNK = CH_T * 128
CHUNK_OUT = CHUNK * H
MAX_SLOTS = 32


def _sc_lookup(e_total):
    n_chunks = e_total // CHUNK
    assert n_chunks * CHUNK == e_total
    assert n_chunks <= NW * MAX_SLOTS
    mesh = plsc.VectorSubcoreMesh(core_axis_name="c", subcore_axis_name="s")

    @functools.partial(
        pl.kernel,
        out_type=jax.ShapeDtypeStruct((e_total * H,), jnp.float32),
        mesh=mesh,
        compiler_params=pltpu.CompilerParams(
            needs_layout_passes=False, use_tc_tiling_on_sc=False),
        scratch_types=[
            pltpu.VMEM((CHUNK,), jnp.int32),
            pltpu.VMEM((CHUNK,), jnp.int32),
            pltpu.VMEM((CHUNK_OUT,), jnp.float32),
            pltpu.VMEM((CHUNK_OUT,), jnp.float32),
            pltpu.VMEM((48,), jnp.float32),
            pltpu.SemaphoreType.DMA,
            pltpu.SemaphoreType.DMA,
            pltpu.SemaphoreType.DMA,
            pltpu.SemaphoreType.DMA,
        ],
    )
    def body(ids_hbm, bias_hbm, out_hbm, ids_v0, ids_v1, out_v0, out_v1,
             bias_v, in_sem0, in_sem1, out_sem0, out_sem1):
        c_ax = lax.axis_index("c")
        s_ax = lax.axis_index("s")
        wid = s_ax * NC + c_ax
        in_sems = (in_sem0, in_sem1)
        out_sems = (out_sem0, out_sem1)
        ids_bufs = (ids_v0, ids_v1)
        out_bufs = (out_v0, out_v1)
        pltpu.sync_copy(bias_hbm, bias_v)

        def ids_copy(ci, b):
            return pltpu.make_async_copy(
                ids_hbm.at[pl.ds(ci * CHUNK, CHUNK)], ids_bufs[b], in_sems[b])

        def out_copy(ci, b):
            return pltpu.make_async_copy(
                out_bufs[b],
                out_hbm.at[pl.ds(ci * CHUNK_OUT, CHUNK_OUT)], out_sems[b])

        ids_copy(wid, 0).start()
        ids_copy(wid + NW, 1).start()

        def two_slots(i, carry):
            for b in range(2):
                j = i * 2 + b
                ci = wid + j * NW

                @pl.when(ci < n_chunks)
                def _():
                    ids_copy(ci, b).wait()
                    @pl.when(j >= 2)
                    def _():
                        out_copy(ci, b).wait()

                    idsb = ids_bufs[b]
                    outb = out_bufs[b]

                    @plsc.parallel_loop(0, CHUNK // 16, unroll=1)
                    def _(k):
                        v8 = idsb[pl.ds(k * 16, 16)] * H
                        off = (k // 8) * 1024 + (k % 8) * 16
                        for h in range(H):
                            outb[pl.ds(off + h * 128, 16)] = plsc.load_gather(
                                bias_v, [v8 + h])

                    out_copy(ci, b).start()

                    @pl.when(ci + 2 * NW < n_chunks)
                    def _():
                        ids_copy(ci + 2 * NW, b).start()
            return carry

        lax.fori_loop(0, MAX_SLOTS // 2, two_slots, 0)
        out_copy(wid, 0).wait()
        out_copy(wid, 1).wait()

    return body


def kernel(band_ids, bias):
    e_total = band_ids.shape[0]
    ids = band_ids.astype(jnp.int32)
    bias_flat = jnp.pad(bias.reshape(-1).astype(jnp.float32), (0, 8))
    flat = _sc_lookup(e_total)(ids, bias_flat)
    tiles = flat.reshape(e_total // 128, H, 128)
    return tiles.transpose(0, 2, 1).reshape(e_total, H)


# speedup vs baseline: 1.0031x; 1.0031x over previous
import functools

import jax
import jax.numpy as jnp
from jax import lax
from jax.experimental import pallas as pl
from jax.experimental.pallas import tpu as pltpu
from jax.experimental.pallas import tpu_sc as plsc

H = 8
NC = 2
NS = 16
NW = NC * NS
CH_T = 25
CHUNK = CH_T * 128
CHUNK_OUT = CHUNK * H
NBUF = 4
MAX_SLOTS = 64


def _sc_lookup(e_total):
    n_chunks = e_total // CHUNK
    assert n_chunks * CHUNK == e_total
    assert n_chunks <= NW * MAX_SLOTS
    mesh = plsc.VectorSubcoreMesh(core_axis_name="c", subcore_axis_name="s")

    @functools.partial(
        pl.kernel,
        out_type=jax.ShapeDtypeStruct((e_total * H,), jnp.float32),
        mesh=mesh,
        compiler_params=pltpu.CompilerParams(
            needs_layout_passes=False, use_tc_tiling_on_sc=False),
        scratch_types=[
            pltpu.VMEM((CHUNK,), jnp.int32),
            pltpu.VMEM((CHUNK,), jnp.int32),
            pltpu.VMEM((CHUNK_OUT,), jnp.float32),
            pltpu.VMEM((CHUNK_OUT,), jnp.float32),
            pltpu.VMEM((CHUNK_OUT,), jnp.float32),
            pltpu.VMEM((CHUNK_OUT,), jnp.float32),
            pltpu.VMEM((48,), jnp.float32),
            pltpu.SemaphoreType.DMA,
            pltpu.SemaphoreType.DMA,
            pltpu.SemaphoreType.DMA,
            pltpu.SemaphoreType.DMA,
            pltpu.SemaphoreType.DMA,
            pltpu.SemaphoreType.DMA,
        ],
    )
    def body(ids_hbm, bias_hbm, out_hbm, ids_v0, ids_v1,
             out_v0, out_v1, out_v2, out_v3,
             bias_v, in_sem0, in_sem1,
             out_sem0, out_sem1, out_sem2, out_sem3):
        c_ax = lax.axis_index("c")
        s_ax = lax.axis_index("s")
        wid = s_ax * NC + c_ax
        in_sems = (in_sem0, in_sem1)
        out_sems = (out_sem0, out_sem1, out_sem2, out_sem3)
        ids_bufs = (ids_v0, ids_v1)
        out_bufs = (out_v0, out_v1, out_v2, out_v3)
        pltpu.sync_copy(bias_hbm, bias_v)

        def ids_copy(ci, b):
            return pltpu.make_async_copy(
                ids_hbm.at[pl.ds(ci * CHUNK, CHUNK)], ids_bufs[b], in_sems[b])

        def out_copy(ci, b):
            return pltpu.make_async_copy(
                out_bufs[b],
                out_hbm.at[pl.ds(ci * CHUNK_OUT, CHUNK_OUT)], out_sems[b])

        ids_copy(wid, 0).start()
        ids_copy(wid + NW, 1).start()

        def two_slots(i, carry):
            for b in range(NBUF):
                j = i * NBUF + b
                ci = wid + j * NW

                @pl.when(ci < n_chunks)
                def _():
                    ids_copy(ci, b % 2).wait()
                    @pl.when(j >= NBUF)
                    def _():
                        out_copy(ci, b).wait()

                    idsb = ids_bufs[b % 2]
                    outb = out_bufs[b]

                    @plsc.parallel_loop(0, CHUNK // 16, unroll=1)
                    def _(k):
                        v8 = idsb[pl.ds(k * 16, 16)] * H
                        off = (k // 8) * 1024 + (k % 8) * 16
                        for h in range(H):
                            outb[pl.ds(off + h * 128, 16)] = plsc.load_gather(
                                bias_v, [v8 + h])

                    out_copy(ci, b).start()

                    @pl.when(ci + 2 * NW < n_chunks)
                    def _():
                        ids_copy(ci + 2 * NW, b % 2).start()
            return carry

        lax.fori_loop(0, MAX_SLOTS // NBUF, two_slots, 0)
        for b in range(NBUF):
            out_copy(wid, b).wait()

    return body


def kernel(band_ids, bias):
    e_total = band_ids.shape[0]
    ids = band_ids.astype(jnp.int32)
    bias_flat = jnp.pad(bias.reshape(-1).astype(jnp.float32), (0, 8))
    flat = _sc_lookup(e_total)(ids, bias_flat)
    tiles = flat.reshape(e_total // 128, H, 128)
    return tiles.transpose(0, 2, 1).reshape(e_total, H)
